# baseline (speedup 1.0000x reference)
import functools

import jax
import jax.numpy as jnp
from jax import lax
from jax.experimental import pallas as pl
from jax.experimental.pallas import tpu as pltpu
from jax.experimental.pallas import tpu_sc as plsc

BATCH = 16384
FIELD = 32
EMB_DIM = 64
NIDX = 3 * FIELD
OUT_D = NIDX * EMB_DIM
TOTAL = BATCH * NIDX
CHUNK = 128
NROWS = TOTAL // CHUNK
NW = 32
ROWS_PER_W = NROWS // NW
GPC = 4
SC_ROWS = CHUNK * GPC
NSC = ROWS_PER_W // GPC
NSTEP = NSC // 2


def _make_kernel():
    mesh = plsc.VectorSubcoreMesh(
        core_axis_name="c", subcore_axis_name="s", num_cores=2, num_subcores=16
    )

    @functools.partial(
        pl.kernel,
        out_type=jax.ShapeDtypeStruct((TOTAL, EMB_DIM), jnp.float32),
        mesh=mesh,
        scratch_types=[
            pltpu.VMEM((ROWS_PER_W, CHUNK), jnp.int32),
            pltpu.VMEM((2, SC_ROWS, EMB_DIM), jnp.float32),
            pltpu.VMEM_SHARED((1000, EMB_DIM), jnp.float32),
            pltpu.SemaphoreType.DMA,
            pltpu.SemaphoreType.DMA,
            pltpu.SemaphoreType.DMA,
            pltpu.SemaphoreType.DMA,
        ],
        compiler_params=pltpu.CompilerParams(use_tc_tiling_on_sc=False),
    )
    def gather_kernel(
        idx_hbm, table_hbm, out_hbm, idx_v, rows_v, tab_sh, g0, g1, s0, s1
    ):
        sid = lax.axis_index("s")
        wid = sid * 2 + lax.axis_index("c")
        base = wid * ROWS_PER_W
        out_base = wid * NSC

        pltpu.sync_copy(table_hbm, tab_sh)
        pltpu.sync_copy(idx_hbm.at[pl.ds(base, ROWS_PER_W), :], idx_v)
        plsc.subcore_barrier()

        def step_body(c, carry):
            buf = rows_v.at[0]
            out_slc = out_hbm.at[pl.ds((out_base + c) * SC_ROWS, SC_ROWS), :]
            handles = [
                pltpu.async_copy(
                    tab_sh.at[idx_v.at[c * GPC + k]],
                    rows_v.at[0, pl.ds(k * CHUNK, CHUNK), :],
                    g0,
                )
                for k in range(GPC)
            ]
            for h in handles:
                h.wait()
            pltpu.sync_copy(buf, out_slc)
            return carry

        lax.fori_loop(0, NSC, step_body, 0, unroll=False)

    return gather_kernel


_gather = _make_kernel()


def kernel(xys, xylens, rgbs, embedding):
    if xys.ndim == 3:
        xys = xys.reshape(xys.shape[0], -1)
    if xylens.ndim == 3:
        xylens = xylens.reshape(xylens.shape[0], -1)
    if rgbs.ndim == 3:
        rgbs = rgbs.reshape(rgbs.shape[0], -1)
    everything = jnp.concatenate((xys, xylens, rgbs), axis=-1)
    idx = everything.reshape(NROWS, CHUNK)
    out = _gather(idx, embedding)
    return out.reshape(xys.shape[0], -1)
